# baseline (speedup 1.0000x reference)
import functools

import jax
import jax.numpy as jnp
from jax import lax
from jax.experimental import pallas as pl
from jax.experimental.pallas import tpu as pltpu
from jax.experimental.pallas import tpu_sc as plsc

D = 32
NC = 2
NS = 16
NW = NC * NS
NBUF = 8
G = 4


def _emb_body(rpw, x_hbm, table_hbm, out_hbm, idx_all, rows, sem_i, sg, sw):
    wid = lax.axis_index("s") * NC + lax.axis_index("c")
    rbase = wid * rpw

    pltpu.async_copy(x_hbm.at[pl.ds(rbase, rpw)], idx_all, sem_i).wait()

    def start_gather(i, b):
        pltpu.async_copy(table_hbm.at[idx_all.at[i]], rows.at[b], sg[b])

    def wait_gather(b):
        pltpu.make_async_copy(
            table_hbm.at[idx_all.at[0]], rows.at[b], sg[b]).wait()

    def start_wb(i, b):
        pltpu.async_copy(rows.at[b], out_hbm.at[rbase + i], sw[b])

    def wait_wb(b):
        pltpu.make_async_copy(rows.at[b], out_hbm.at[0], sw[b]).wait()

    for i in range(G):
        start_gather(i, i)
    for i in range(NBUF - G):
        wait_gather(i)
        start_wb(i, i)
        start_gather(i + G, i + G)
    h = NBUF - G

    def steady(jj, carry):
        for k in range(NBUF):
            i = h + NBUF * jj + k
            b = (h + k) % NBUF
            wait_gather(b)
            start_wb(i, b)
            b2 = (b + G) % NBUF
            wait_wb(b2)
            start_gather(i + G, b2)
        return carry

    lax.fori_loop(0, (rpw - h - G) // NBUF, steady, 0)

    for i in range(rpw - G, rpw):
        b = i % NBUF
        wait_gather(b)
        start_wb(i, b)
    for i in range(rpw - NBUF, rpw):
        wait_wb(i % NBUF)


@jax.jit
def kernel(x, table):
    batch, hist = x.shape
    rpw = batch // NW
    mesh = plsc.VectorSubcoreMesh(core_axis_name="c", subcore_axis_name="s")
    fn = pl.kernel(
        functools.partial(_emb_body, rpw),
        mesh=mesh,
        out_type=jax.ShapeDtypeStruct((batch, hist, D), jnp.float32),
        scratch_types=[
            pltpu.VMEM((rpw, hist), jnp.int32),
            pltpu.VMEM((NBUF, hist, D), jnp.float32),
            pltpu.SemaphoreType.DMA,
            [pltpu.SemaphoreType.DMA] * NBUF,
            [pltpu.SemaphoreType.DMA] * NBUF,
        ],
        compiler_params=pltpu.CompilerParams(use_tc_tiling_on_sc=False),
    )
    return fn(x, table)


# speedup vs baseline: 1.0004x; 1.0004x over previous
import functools

import jax
import jax.numpy as jnp
from jax import lax
from jax.experimental import pallas as pl
from jax.experimental.pallas import tpu as pltpu
from jax.experimental.pallas import tpu_sc as plsc

D = 32
NC = 2
NS = 16
NW = NC * NS
NBUF = 8
G = 4


def _emb_body(rpw, x_hbm, table_hbm, out_hbm, idx_all, rows, sem_i, sg, sw):
    wid = lax.axis_index("s") * NC + lax.axis_index("c")
    rbase = wid * rpw

    pltpu.async_copy(x_hbm.at[pl.ds(rbase, rpw)], idx_all, sem_i).wait()

    def start_gather(i, b):
        pltpu.async_copy(table_hbm.at[idx_all.at[i]], rows.at[b], sg[b])

    def wait_gather(b):
        pltpu.make_async_copy(
            table_hbm.at[idx_all.at[0]], rows.at[b], sg[b]).wait()

    hist = idx_all.shape[1]

    def start_wb(i, b):
        pltpu.async_copy(
            rows.at[b], out_hbm.at[pl.ds((rbase + i) * hist, hist)], sw[b])

    def wait_wb(b):
        pltpu.make_async_copy(
            rows.at[b], out_hbm.at[pl.ds(0, hist)], sw[b]).wait()

    for i in range(G):
        start_gather(i, i)
    for i in range(NBUF - G):
        wait_gather(i)
        start_wb(i, i)
        start_gather(i + G, i + G)
    h = NBUF - G

    def steady(jj, carry):
        for k in range(NBUF):
            i = h + NBUF * jj + k
            b = (h + k) % NBUF
            wait_gather(b)
            start_wb(i, b)
            b2 = (b + G) % NBUF
            wait_wb(b2)
            start_gather(i + G, b2)
        return carry

    lax.fori_loop(0, (rpw - h - G) // NBUF, steady, 0)

    for i in range(rpw - G, rpw):
        b = i % NBUF
        wait_gather(b)
        start_wb(i, b)
    for i in range(rpw - NBUF, rpw):
        wait_wb(i % NBUF)


@jax.jit
def kernel(x, table):
    batch, hist = x.shape
    rpw = batch // NW
    mesh = plsc.VectorSubcoreMesh(core_axis_name="c", subcore_axis_name="s")
    fn = pl.kernel(
        functools.partial(_emb_body, rpw),
        mesh=mesh,
        out_type=jax.ShapeDtypeStruct((batch * hist, D), jnp.float32),
        scratch_types=[
            pltpu.VMEM((rpw, hist), jnp.int32),
            pltpu.VMEM((NBUF, hist, D), jnp.float32),
            pltpu.SemaphoreType.DMA,
            [pltpu.SemaphoreType.DMA] * NBUF,
            [pltpu.SemaphoreType.DMA] * NBUF,
        ],
        compiler_params=pltpu.CompilerParams(use_tc_tiling_on_sc=False),
    )
    return fn(x, table).reshape(batch, hist, D)
